# baseline (speedup 1.0000x reference)
import jax
import jax.numpy as jnp
from jax.experimental import pallas as pl
from jax.experimental.pallas import tpu as pltpu

NB = 16
NA = 3
NC = 80
G = 76
C = NC + 5
P = G * G
STRIDE = 608.0 / G
ANCHOR_W = (10.0, 16.0, 33.0)
ANCHOR_H = (13.0, 30.0, 23.0)


def _decode_body(x_ref, o_ref):
    lane = jax.lax.broadcasted_iota(jnp.int32, (G, C), 1)
    is_wh = (lane == 2) | (lane == 3)
    rowj = jax.lax.broadcasted_iota(jnp.int32, (G, C), 0).astype(jnp.float32)
    for a in range(NA):
        scale = jnp.where(
            lane <= 1, STRIDE,
            jnp.where(lane == 2, ANCHOR_W[a],
                      jnp.where(lane == 3, ANCHOR_H[a], 1.0)))
        for i in range(G):
            t = x_ref[0, a * C:(a + 1) * C, i, :].T
            sig = 0.5 + 0.5 * jnp.tanh(t * 0.5)
            e = jnp.exp(t)
            val = jnp.where(is_wh, e, sig)
            bias = jnp.where(lane == 0, rowj * STRIDE,
                             jnp.where(lane == 1, i * STRIDE, 0.0))
            o_ref[0, a * P + i * G:a * P + (i + 1) * G, :] = val * scale + bias


def kernel(x):
    return pl.pallas_call(
        _decode_body,
        grid=(NB,),
        in_specs=[pl.BlockSpec((1, NA * C, G, G), lambda b: (b, 0, 0, 0))],
        out_specs=pl.BlockSpec((1, NA * P, C), lambda b: (b, 0, 0)),
        out_shape=jax.ShapeDtypeStruct((NB, NA * P, C), jnp.float32),
        compiler_params=pltpu.CompilerParams(
            dimension_semantics=("arbitrary",),
        ),
    )(x)
